# baseline (speedup 1.0000x reference)
import functools

import jax
import jax.numpy as jnp
from jax import lax
from jax.experimental import pallas as pl
from jax.experimental.pallas import tpu as pltpu
from jax.experimental.pallas import tpu_sc as plsc

_B = 8192
_D = 1024
_NW = 32
_BPW = _B // _NW
_C = 16
_NCH = _BPW // _C
_NBUF = 7


def _build():
    mesh = plsc.VectorSubcoreMesh(core_axis_name="c", subcore_axis_name="s")

    @functools.partial(
        pl.kernel,
        mesh=mesh,
        out_type=jax.ShapeDtypeStruct((_B, _D), jnp.float32),
        scratch_types=[
            pltpu.VMEM((_NCH, _C), jnp.int32),
            pltpu.VMEM((_NBUF, _C, _D), jnp.float32),
            pltpu.SemaphoreType.DMA,
            pltpu.SemaphoreType.DMA,
        ],
    )
    def gather_kernel(idx_hbm, table_hbm, out_hbm, idx_v, bufs, gsem, osem):
        wid = lax.axis_index("s") * 2 + lax.axis_index("c")
        base = wid * _BPW
        pltpu.sync_copy(idx_hbm.at[wid], idx_v)

        gathers = [None] * _NCH
        outs = [None] * _NCH
        for g in range(_NBUF):
            gathers[g] = pltpu.async_copy(
                table_hbm.at[idx_v.at[g]], bufs.at[g % _NBUF], gsem)
        for g in range(_NCH):
            gathers[g].wait()
            outs[g] = pltpu.async_copy(
                bufs.at[g % _NBUF],
                out_hbm.at[pl.ds(base + g * _C, _C)], osem)
            nxt = g + _NBUF
            if nxt < _NCH:
                outs[g].wait()
                gathers[nxt] = pltpu.async_copy(
                    table_hbm.at[idx_v.at[nxt]], bufs.at[nxt % _NBUF], gsem)
        for g in range(_NCH - _NBUF, _NCH):
            outs[g].wait()

    return gather_kernel


_GATHER = _build()


def kernel(workpiece_idx, pos_encoding):
    idx = workpiece_idx.astype(jnp.int32).reshape(_NW, _NCH, _C)
    out = _GATHER(idx, pos_encoding)
    return out.reshape(workpiece_idx.shape + (_D,))


# speedup vs baseline: 1.0133x; 1.0133x over previous
import functools

import jax
import jax.numpy as jnp
from jax import lax
from jax.experimental import pallas as pl
from jax.experimental.pallas import tpu as pltpu
from jax.experimental.pallas import tpu_sc as plsc

_N = 4
_S = 2048
_D = 1024
_NW = 32
_BPW = (_N * _S) // _NW
_WPR = _S // _BPW
_C = 32
_NCH = _BPW // _C
_NBUF = 3


def _build():
    mesh = plsc.VectorSubcoreMesh(core_axis_name="c", subcore_axis_name="s")

    @functools.partial(
        pl.kernel,
        mesh=mesh,
        out_type=jax.ShapeDtypeStruct((_N, _S, _D), jnp.float32),
        scratch_types=[
            pltpu.VMEM((_BPW,), jnp.int32),
            pltpu.VMEM((_NBUF, _C, _D), jnp.float32),
            pltpu.SemaphoreType.DMA,
            pltpu.SemaphoreType.DMA,
        ],
    )
    def gather_kernel(idx_hbm, table_hbm, out_hbm, idx_v, bufs, gsem, osem):
        wid = lax.axis_index("s") * 2 + lax.axis_index("c")
        row = wid // _WPR
        col = (wid % _WPR) * _BPW
        pltpu.sync_copy(idx_hbm.at[row, pl.ds(col, _BPW)], idx_v)

        gathers = [None] * _NCH
        outs = [None] * _NCH
        for g in range(_NBUF):
            gathers[g] = pltpu.async_copy(
                table_hbm.at[idx_v.at[pl.ds(g * _C, _C)]],
                bufs.at[g % _NBUF], gsem)
        for g in range(_NCH):
            gathers[g].wait()
            outs[g] = pltpu.async_copy(
                bufs.at[g % _NBUF],
                out_hbm.at[row, pl.ds(col + g * _C, _C)], osem)
            nxt = g + _NBUF
            if nxt < _NCH:
                outs[g].wait()
                gathers[nxt] = pltpu.async_copy(
                    table_hbm.at[idx_v.at[pl.ds(nxt * _C, _C)]],
                    bufs.at[nxt % _NBUF], gsem)
        for g in range(_NCH - _NBUF, _NCH):
            outs[g].wait()

    return gather_kernel


_GATHER = _build()


def kernel(workpiece_idx, pos_encoding):
    return _GATHER(workpiece_idx.astype(jnp.int32), pos_encoding)
